# baseline (speedup 1.0000x reference)
import functools

import jax
import jax.numpy as jnp
from jax import lax
from jax.experimental import pallas as pl
from jax.experimental.pallas import tpu as pltpu
from jax.experimental.pallas import tpu_sc as plsc

BATCH = 16384
NROWS = 1000000
EMBED_DIM = 64
OUT_LANES = 128
NUM_CORES = 2
NUM_SUBCORES = 16
NUM_WORKERS = NUM_CORES * NUM_SUBCORES
B_PER_W = BATCH // NUM_WORKERS
CH = 224
LANES = 16
NGROUPS = B_PER_W // LANES
SCAT = 128


@functools.cache
def _build():
    mesh = plsc.VectorSubcoreMesh(
        core_axis_name="c", subcore_axis_name="s",
        num_cores=NUM_CORES, num_subcores=NUM_SUBCORES)

    @functools.partial(
        pl.kernel,
        mesh=mesh,
        compiler_params=pltpu.CompilerParams(needs_layout_passes=False),
        out_type=(
            jax.ShapeDtypeStruct((BATCH, OUT_LANES), jnp.float32),
            jax.ShapeDtypeStruct((BATCH, OUT_LANES), jnp.float32),
        ),
        scratch_types=[
            pltpu.VMEM((B_PER_W,), jnp.int32),
            pltpu.VMEM((CH, EMBED_DIM), jnp.float32),
            pltpu.VMEM((CH, EMBED_DIM), jnp.float32),
            pltpu.VMEM((B_PER_W, OUT_LANES), jnp.float32),
            pltpu.SemaphoreType.DMA,
            pltpu.SemaphoreType.DMA,
            pltpu.SemaphoreType.DMA,
        ],
    )
    def gather2(squ_hbm, sqi_hbm, ut_hbm, it_hbm,
                out_u, out_i, sq_v, cb0, cb1, stage_v,
                csem_a, csem_b, wsem):
        wid = lax.axis_index("s") * NUM_CORES + lax.axis_index("c")
        base = wid * B_PER_W

        def chunk_start(c):
            return pl.multiple_of(jnp.minimum(lo + c * CH, NROWS - CH), 8)

        def fire_chunk(tbl, c, cb, sem):
            pltpu.async_copy(tbl.at[pl.ds(chunk_start(c), CH)], cb, sem)

        def drain_c(tbl, sem):
            pltpu.make_async_copy(tbl.at[pl.ds(0, CH)], cb0, sem).wait()

        def extract(cb, start_c, chunk_end, g_start, prev_end):
            def gbody(gi, ng):
                v = sq_v[pl.ds(gi * LANES, LANES)]
                m = (v >= prev_end) & (v < chunk_end)
                npick = plsc.all_reduce_population_count(m)[0]

                @pl.when(npick > 0)
                def _():
                    local = v - start_c
                    mi = m.astype(jnp.int32)
                    for lane in range(LANES):
                        @pl.when(mi[lane] > 0)
                        def _(lane=lane):
                            r = local[lane]
                            row = gi * LANES + lane
                            for k in range(EMBED_DIM // LANES):
                                stage_v[row, pl.ds(k * LANES, LANES)] = (
                                    cb[r, pl.ds(k * LANES, LANES)])

                vmax = lax.reduce_max(v, (0,))
                return ng + (vmax < chunk_end).astype(jnp.int32)

            return lax.fori_loop(g_start, NGROUPS, gbody, g_start)

        for t, (sq_hbm, tbl, out) in enumerate((
                (squ_hbm, ut_hbm, out_u),
                (sqi_hbm, it_hbm, out_i))):
            pltpu.sync_copy(sq_hbm.at[pl.ds(base, B_PER_W)], sq_v)
            if t == 1:
                pltpu.make_async_copy(
                    out_u.at[pl.ds(0, B_PER_W)], stage_v, wsem).wait()

            lo = sq_v[pl.ds(0, LANES)][0] & ~7
            hi = sq_v[pl.ds(B_PER_W - LANES, LANES)][LANES - 1]
            m_chunks = lax.div(hi - lo, jnp.int32(CH)) + 1

            fire_chunk(tbl, 0, cb0, csem_a)

            def pair_body(i, carry):
                g, prev_end = carry
                c0 = 2 * i
                c1 = 2 * i + 1
                has_odd = c1 < m_chunks

                @pl.when(has_odd)
                def _():
                    fire_chunk(tbl, c1, cb1, csem_b)
                drain_c(tbl, csem_a)
                even_end = chunk_start(c0) + CH
                g = extract(cb0, chunk_start(c0), even_end, g, prev_end)

                @pl.when(has_odd)
                def _():
                    fire_chunk(tbl, c1 + 1, cb0, csem_a)
                    drain_c(tbl, csem_b)
                odd_end = jnp.where(has_odd, chunk_start(c1) + CH, even_end)
                g = extract(cb1, jnp.where(has_odd, chunk_start(c1), even_end),
                            odd_end, g, even_end)
                return (g, odd_end)

            n_pairs = lax.div(m_chunks + 1, jnp.int32(2))
            lax.fori_loop(0, n_pairs, pair_body,
                          (jnp.int32(0), jnp.int32(0)))
            @pl.when((m_chunks > 1) & (lax.rem(m_chunks, jnp.int32(2)) == 0))
            def _():
                drain_c(tbl, csem_a)

            pltpu.async_copy(stage_v, out.at[pl.ds(base, B_PER_W)], wsem)
        pltpu.make_async_copy(out_u.at[pl.ds(0, B_PER_W)],
                              stage_v, wsem).wait()

    return gather2


def kernel(query_users, query_items, user_table, item_table):
    if query_users.ndim > 1:
        query_users = jnp.squeeze(query_users, axis=0)
    if query_items.ndim > 1:
        query_items = jnp.squeeze(query_items, axis=0)
    qu = query_users.astype(jnp.int32)
    qi = query_items.astype(jnp.int32)
    pos = lax.iota(jnp.int32, BATCH)
    squ, ordu = lax.sort_key_val(qu, pos)
    sqi, ordi = lax.sort_key_val(qi, pos)
    u128, i128 = _build()(squ, sqi, user_table, item_table)
    _, invu = lax.sort_key_val(ordu, pos)
    _, invi = lax.sort_key_val(ordi, pos)
    return (jnp.take(u128, invu, axis=0)[:, :EMBED_DIM],
            jnp.take(i128, invi, axis=0)[:, :EMBED_DIM])


# speedup vs baseline: 1.0492x; 1.0492x over previous
import functools

import jax
import jax.numpy as jnp
from jax import lax
from jax.experimental import pallas as pl
from jax.experimental.pallas import tpu as pltpu
from jax.experimental.pallas import tpu_sc as plsc

BATCH = 16384
NROWS = 1000000
EMBED_DIM = 64
PAIR_LANES = 2 * EMBED_DIM
NUM_CORES = 2
NUM_SUBCORES = 16
NUM_WORKERS = NUM_CORES * NUM_SUBCORES
B_PER_W = BATCH // NUM_WORKERS
BB = 256
LANES = 16


@functools.cache
def _build():
    mesh = plsc.VectorSubcoreMesh(
        core_axis_name="c", subcore_axis_name="s",
        num_cores=NUM_CORES, num_subcores=NUM_SUBCORES)

    @functools.partial(
        pl.kernel,
        mesh=mesh,
        out_type=(
            jax.ShapeDtypeStruct((BATCH, EMBED_DIM), jnp.float32),
            jax.ShapeDtypeStruct((BATCH, EMBED_DIM), jnp.float32),
        ),
        scratch_types=[
            pltpu.VMEM((B_PER_W + LANES,), jnp.int32),
            pltpu.VMEM((B_PER_W,), jnp.int32),
            pltpu.VMEM((BB, PAIR_LANES), jnp.float32),
            pltpu.VMEM((BB, EMBED_DIM), jnp.float32),
            pltpu.SemaphoreType.DMA,
            pltpu.SemaphoreType.DMA,
        ],
    )
    def gather2(qu_hbm, qi_hbm, ut_hbm, it_hbm, out_u, out_i,
                idx_v, pidx_v, rows_v, stage_v, gsem, wsem):
        wid = lax.axis_index("s") * NUM_CORES + lax.axis_index("c")
        base = wid * B_PER_W

        for tbl, q_hbm, out in ((ut_hbm, qu_hbm, out_u),
                                (it_hbm, qi_hbm, out_i)):
            pltpu.sync_copy(q_hbm.at[pl.ds(base, B_PER_W)],
                            idx_v.at[pl.ds(0, B_PER_W)])
            for s in range(B_PER_W // LANES):
                pidx_v[pl.ds(s * LANES, LANES)] = lax.shift_right_logical(
                    idx_v[pl.ds(s * LANES, LANES)], 1)

            for b in range(B_PER_W // BB):
                pltpu.async_copy(
                    tbl.at[pidx_v.at[pl.ds(b * BB, BB)]], rows_v, gsem).wait()

                def select(i, _):
                    q = idx_v[pl.ds(b * BB + i, LANES)][0]

                    @pl.when((q & 1) == 0)
                    def _():
                        for k in range(EMBED_DIM // LANES):
                            stage_v[i, pl.ds(k * LANES, LANES)] = (
                                rows_v[i, pl.ds(k * LANES, LANES)])

                    @pl.when((q & 1) == 1)
                    def _():
                        for k in range(EMBED_DIM // LANES):
                            stage_v[i, pl.ds(k * LANES, LANES)] = (
                                rows_v[i, pl.ds(EMBED_DIM + k * LANES, LANES)])
                    return ()
                lax.fori_loop(0, BB, select, ())

                pltpu.async_copy(stage_v,
                                 out.at[pl.ds(base + b * BB, BB)], wsem)
                pltpu.make_async_copy(out_u.at[pl.ds(0, BB)],
                                      stage_v, wsem).wait()

    return gather2


def kernel(query_users, query_items, user_table, item_table):
    if query_users.ndim > 1:
        query_users = jnp.squeeze(query_users, axis=0)
    if query_items.ndim > 1:
        query_items = jnp.squeeze(query_items, axis=0)
    return _build()(query_users.astype(jnp.int32),
                    query_items.astype(jnp.int32),
                    user_table.reshape(NROWS // 2, PAIR_LANES),
                    item_table.reshape(NROWS // 2, PAIR_LANES))
